# baseline (speedup 1.0000x reference)
import jax
import jax.numpy as jnp
from jax import lax
from jax.experimental import pallas as pl
from jax.experimental.pallas import tpu as pltpu
from jax.experimental.pallas import tpu_sc as plsc

N_NODES = 10000
N_EDGES = 320000
D = 128
DH = D // 2

NC = 2
NS = 16
CHUNK = 125
CPT = 160
NBUF = 4
NR = CPT // NBUF
WB = 200
CP_TILES = 10
ROWS_PER_CP = N_NODES // CP_TILES
ZROWS = N_NODES // NS
WROWS = N_NODES // NS


def _make_sc_agg(with_counts: bool):
  mesh = plsc.VectorSubcoreMesh(core_axis_name="c", subcore_axis_name="s")
  out_type = [jax.ShapeDtypeStruct((N_NODES, D), jnp.float32)]
  scratch = [
      pltpu.VMEM((CPT, CHUNK), jnp.int32),
      pltpu.VMEM((CPT, CHUNK), jnp.int32),
      [pltpu.VMEM((CHUNK, DH), jnp.float32) for _ in range(NBUF)],
      pltpu.VMEM((WB, DH), jnp.float32),
      pltpu.VMEM_SHARED((N_NODES, DH), jnp.float32),
      [pltpu.SemaphoreType.DMA for _ in range(NBUF)],
      [pltpu.SemaphoreType.DMA for _ in range(NBUF)],
  ]
  if with_counts:
    out_type.append(jax.ShapeDtypeStruct((N_NODES,), jnp.float32))
    scratch += [
        pltpu.VMEM((128,), jnp.float32),
        pltpu.VMEM((1008,), jnp.float32),
        pltpu.VMEM_SHARED((N_NODES,), jnp.float32),
        pltpu.SemaphoreType.DMA,
    ]

  def body(*refs):
    if with_counts:
      (xr_hbm, idx_hbm, out_hbm, cnt_hbm,
       srcv, dstv, msgs, wb, acc, gsem, ssem, ones, zc, cacc, csem) = refs
    else:
      (xr_hbm, idx_hbm, out_hbm,
       srcv, dstv, msgs, wb, acc, gsem, ssem) = refs

    c = lax.axis_index("c")
    s = lax.axis_index("s")

    pltpu.async_copy(idx_hbm.at[c, s], srcv, gsem[0])
    pltpu.async_copy(idx_hbm.at[2, s], dstv, gsem[1])

    z16 = jnp.zeros((16,), jnp.float32)

    def zrow(r, carry):
      for k in range(DH // 16):
        wb[r, pl.ds(k * 16, 16)] = z16
      return carry

    lax.fori_loop(0, WB, zrow, 0)

    off = 0
    while off < ZROWS:
      n = min(WB, ZROWS - off)
      pltpu.sync_copy(wb.at[pl.ds(0, n)],
                      acc.at[pl.ds(s * ZROWS + off, n)])
      off += n

    if with_counts:
      for k in range(8):
        ones[pl.ds(k * 16, 16)] = jnp.ones((16,), jnp.float32)
      for k in range(63):
        zc[pl.ds(k * 16, 16)] = z16

      @pl.when((c == 0) & (s < CP_TILES))
      def _():
        pltpu.sync_copy(zc.at[pl.ds(0, ROWS_PER_CP)],
                        cacc.at[pl.ds(s * ROWS_PER_CP, ROWS_PER_CP)])

    pltpu.make_async_copy(idx_hbm.at[c, s], srcv, gsem[0]).wait()
    pltpu.make_async_copy(idx_hbm.at[2, s], dstv, gsem[1]).wait()
    plsc.subcore_barrier()

    do_counts = with_counts

    def step(i, carry):
      for b in range(NBUF):
        j = i * NBUF + b

        @pl.when(i > 0)
        def _():
          pltpu.make_async_copy(
              msgs[b], acc.at[dstv.at[j - NBUF]], ssem[b]).wait()

        pltpu.async_copy(xr_hbm.at[srcv.at[j]], msgs[b], gsem[b])

      for b in range(NBUF):
        j = i * NBUF + b
        pltpu.make_async_copy(xr_hbm.at[srcv.at[j]], msgs[b], gsem[b]).wait()
        pltpu.async_copy(msgs[b], acc.at[dstv.at[j]], ssem[b], add=True)
        if do_counts:
          @pl.when(c == 0)
          def _():
            pltpu.async_copy(ones.at[pl.ds(0, CHUNK)], cacc.at[dstv.at[j]],
                             csem, add=True)
      return carry

    lax.fori_loop(0, NR, step, 0)

    for b in range(NBUF):
      pltpu.make_async_copy(
          msgs[b], acc.at[dstv.at[CPT - NBUF + b]], ssem[b]).wait()
    if do_counts:
      @pl.when(c == 0)
      def _():
        def cdrain(j, carry):
          pltpu.make_async_copy(ones.at[pl.ds(0, CHUNK)],
                                cacc.at[dstv.at[j]], csem).wait()
          return carry
        lax.fori_loop(0, CPT, cdrain, 0)

    plsc.subcore_barrier()

    off = 0
    while off < WROWS:
      n = min(WB, WROWS - off)
      r0 = s * WROWS + off
      pltpu.sync_copy(acc.at[pl.ds(r0, n)], wb.at[pl.ds(0, n)])
      pltpu.sync_copy(wb.at[pl.ds(0, n)],
                      out_hbm.at[pl.ds(r0, n), pl.ds(c * DH, DH)])
      off += n

    if with_counts:
      @pl.when((c == 0) & (s < CP_TILES))
      def _():
        pltpu.sync_copy(cacc.at[pl.ds(s * ROWS_PER_CP, ROWS_PER_CP)],
                        zc.at[pl.ds(0, ROWS_PER_CP)])
        pltpu.sync_copy(zc.at[pl.ds(0, ROWS_PER_CP)],
                        cnt_hbm.at[pl.ds(s * ROWS_PER_CP, ROWS_PER_CP)])

  return pl.kernel(
      body, out_type=out_type, mesh=mesh, scratch_types=scratch,
      compiler_params=pltpu.CompilerParams(use_tc_tiling_on_sc=False))


_sc_agg_counts = _make_sc_agg(True)
_sc_agg = _make_sc_agg(False)

def _dense_body(p_ref, cnt_ref, x_ref, wl_ref, b_ref, wr_ref, o_ref):
  cnt = jnp.maximum(cnt_ref[...], 1.0)
  dn0 = (((0,), (0,)), ((), ()))
  cnt_col = lax.dot_general(cnt, jnp.ones((1, 1), jnp.float32), dn0,
                            preferred_element_type=jnp.float32,
                            precision=lax.Precision.HIGHEST)
  agg = p_ref[...] / cnt_col
  dn = (((1,), (1,)), ((), ()))
  h = lax.dot_general(agg, wl_ref[...], dn, preferred_element_type=jnp.float32)
  h = h + lax.dot_general(x_ref[...], wr_ref[...], dn,
                          preferred_element_type=jnp.float32)
  o_ref[...] = jnp.maximum(h + b_ref[...], 0.0)


_dense = pl.pallas_call(
    _dense_body,
    in_specs=[
        pl.BlockSpec((N_NODES, D), lambda: (0, 0)),
        pl.BlockSpec((1, N_NODES), lambda: (0, 0)),
        pl.BlockSpec((N_NODES, D), lambda: (0, 0)),
        pl.BlockSpec((D, D), lambda: (0, 0)),
        pl.BlockSpec((1, D), lambda: (0, 0)),
        pl.BlockSpec((D, D), lambda: (0, 0)),
    ],
    out_specs=pl.BlockSpec((N_NODES, D), lambda: (0, 0)),
    out_shape=jax.ShapeDtypeStruct((N_NODES, D), jnp.float32),
)


@jax.jit
def kernel(x, edge_index, W1_l, b1_l, W1_r, W2_l, b2_l, W2_r):
  e3 = edge_index.astype(jnp.int32).reshape(2, NS, CPT, CHUNK)
  s2 = e3[0:1] * 2
  idx = jnp.concatenate([s2, s2 + 1, e3[1:2]], axis=0)
  xr = x.reshape(2 * N_NODES, DH)

  p1, cnt = _sc_agg_counts(xr, idx)
  cnt2 = cnt.reshape(1, N_NODES)
  h = _dense(p1, cnt2, x, W1_l, b1_l.reshape(1, D), W1_r)
  (p2,) = _sc_agg(h.reshape(2 * N_NODES, DH), idx)
  return _dense(p2, cnt2, h, W2_l, b2_l.reshape(1, D), W2_r)


# speedup vs baseline: 1.0194x; 1.0194x over previous
import jax
import jax.numpy as jnp
from jax import lax
from jax.experimental import pallas as pl
from jax.experimental.pallas import tpu as pltpu
from jax.experimental.pallas import tpu_sc as plsc

N_NODES = 10000
N_EDGES = 320000
D = 128
DH = D // 2

NC = 2
NS = 16
CHUNK = 125
CPT = 160
NBUF = 4
NR = CPT // NBUF
WB = 200
CP_TILES = 10
ROWS_PER_CP = N_NODES // CP_TILES
ZROWS = N_NODES // NS
WROWS = N_NODES // NS


def _make_sc_agg(with_counts: bool):
  mesh = plsc.VectorSubcoreMesh(core_axis_name="c", subcore_axis_name="s")
  out_type = [jax.ShapeDtypeStruct((N_NODES, D), jnp.float32)]
  scratch = [
      pltpu.VMEM((CPT, CHUNK), jnp.int32),
      pltpu.VMEM((CPT, CHUNK), jnp.int32),
      [pltpu.VMEM((CHUNK, DH), jnp.float32) for _ in range(NBUF)],
      pltpu.VMEM((WB, DH), jnp.float32),
      pltpu.VMEM_SHARED((N_NODES, DH), jnp.float32),
      [pltpu.SemaphoreType.DMA for _ in range(NBUF)],
      [pltpu.SemaphoreType.DMA for _ in range(NBUF)],
  ]
  if with_counts:
    out_type.append(jax.ShapeDtypeStruct((NC, N_NODES), jnp.float32))
    scratch += [
        pltpu.VMEM((128,), jnp.float32),
        pltpu.VMEM((1008,), jnp.float32),
        pltpu.VMEM_SHARED((N_NODES,), jnp.float32),
        pltpu.SemaphoreType.DMA,
    ]

  def body(*refs):
    if with_counts:
      (xr_hbm, idx_hbm, out_hbm, cnt_hbm,
       srcv, dstv, msgs, wb, acc, gsem, ssem, ones, zc, cacc, csem) = refs
    else:
      (xr_hbm, idx_hbm, out_hbm,
       srcv, dstv, msgs, wb, acc, gsem, ssem) = refs

    c = lax.axis_index("c")
    s = lax.axis_index("s")

    pltpu.async_copy(idx_hbm.at[c, s], srcv, gsem[0])
    pltpu.async_copy(idx_hbm.at[2, s], dstv, gsem[1])

    z16 = jnp.zeros((16,), jnp.float32)

    def zrow(r, carry):
      for k in range(DH // 16):
        wb[r, pl.ds(k * 16, 16)] = z16
      return carry

    lax.fori_loop(0, WB, zrow, 0)

    off = 0
    while off < ZROWS:
      n = min(WB, ZROWS - off)
      pltpu.sync_copy(wb.at[pl.ds(0, n)],
                      acc.at[pl.ds(s * ZROWS + off, n)])
      off += n

    if with_counts:
      for k in range(8):
        ones[pl.ds(k * 16, 16)] = jnp.ones((16,), jnp.float32)
      for k in range(63):
        zc[pl.ds(k * 16, 16)] = z16

      @pl.when(s < CP_TILES)
      def _():
        pltpu.sync_copy(zc.at[pl.ds(0, ROWS_PER_CP)],
                        cacc.at[pl.ds(s * ROWS_PER_CP, ROWS_PER_CP)])

    pltpu.make_async_copy(idx_hbm.at[c, s], srcv, gsem[0]).wait()
    pltpu.make_async_copy(idx_hbm.at[2, s], dstv, gsem[1]).wait()
    plsc.subcore_barrier()

    do_counts = with_counts

    def step(i, carry):
      for b in range(NBUF):
        j = i * NBUF + b

        @pl.when(i > 0)
        def _():
          pltpu.make_async_copy(
              msgs[b], acc.at[dstv.at[j - NBUF]], ssem[b]).wait()

        pltpu.async_copy(xr_hbm.at[srcv.at[j]], msgs[b], gsem[b])

      for b in range(NBUF):
        j = i * NBUF + b
        pltpu.make_async_copy(xr_hbm.at[srcv.at[j]], msgs[b], gsem[b]).wait()
        pltpu.async_copy(msgs[b], acc.at[dstv.at[j]], ssem[b], add=True)
        if do_counts:
          @pl.when((j < CPT // 2) == (c == 0))
          def _():
            pltpu.async_copy(ones.at[pl.ds(0, CHUNK)], cacc.at[dstv.at[j]],
                             csem, add=True)
      return carry

    lax.fori_loop(0, NR, step, 0)

    for b in range(NBUF):
      pltpu.make_async_copy(
          msgs[b], acc.at[dstv.at[CPT - NBUF + b]], ssem[b]).wait()
    if do_counts:
      def cdrain(j, carry):
        pltpu.make_async_copy(ones.at[pl.ds(0, CHUNK)],
                              cacc.at[dstv.at[j]], csem).wait()
        return carry
      lax.fori_loop(c * (CPT // 2), (c + 1) * (CPT // 2), cdrain, 0)

    plsc.subcore_barrier()

    off = 0
    while off < WROWS:
      n = min(WB, WROWS - off)
      r0 = s * WROWS + off
      pltpu.sync_copy(acc.at[pl.ds(r0, n)], wb.at[pl.ds(0, n)])
      pltpu.sync_copy(wb.at[pl.ds(0, n)],
                      out_hbm.at[pl.ds(r0, n), pl.ds(c * DH, DH)])
      off += n

    if with_counts:
      @pl.when(s < CP_TILES)
      def _():
        pltpu.sync_copy(cacc.at[pl.ds(s * ROWS_PER_CP, ROWS_PER_CP)],
                        zc.at[pl.ds(0, ROWS_PER_CP)])
        pltpu.sync_copy(zc.at[pl.ds(0, ROWS_PER_CP)],
                        cnt_hbm.at[c, pl.ds(s * ROWS_PER_CP, ROWS_PER_CP)])

  return pl.kernel(
      body, out_type=out_type, mesh=mesh, scratch_types=scratch,
      compiler_params=pltpu.CompilerParams(use_tc_tiling_on_sc=False))


_sc_agg_counts = _make_sc_agg(True)
_sc_agg = _make_sc_agg(False)

def _dense_body(p_ref, cnt_ref, x_ref, wl_ref, b_ref, wr_ref, o_ref):
  dn0 = (((0,), (0,)), ((), ()))
  cnt_col = lax.dot_general(cnt_ref[...], jnp.ones((2, 1), jnp.float32),
                            dn0, preferred_element_type=jnp.float32,
                            precision=lax.Precision.HIGHEST)
  agg = p_ref[...] / jnp.maximum(cnt_col, 1.0)
  dn = (((1,), (1,)), ((), ()))
  h = lax.dot_general(agg, wl_ref[...], dn, preferred_element_type=jnp.float32)
  h = h + lax.dot_general(x_ref[...], wr_ref[...], dn,
                          preferred_element_type=jnp.float32)
  o_ref[...] = jnp.maximum(h + b_ref[...], 0.0)


_dense = pl.pallas_call(
    _dense_body,
    in_specs=[
        pl.BlockSpec((N_NODES, D), lambda: (0, 0)),
        pl.BlockSpec((NC, N_NODES), lambda: (0, 0)),
        pl.BlockSpec((N_NODES, D), lambda: (0, 0)),
        pl.BlockSpec((D, D), lambda: (0, 0)),
        pl.BlockSpec((1, D), lambda: (0, 0)),
        pl.BlockSpec((D, D), lambda: (0, 0)),
    ],
    out_specs=pl.BlockSpec((N_NODES, D), lambda: (0, 0)),
    out_shape=jax.ShapeDtypeStruct((N_NODES, D), jnp.float32),
)


@jax.jit
def kernel(x, edge_index, W1_l, b1_l, W1_r, W2_l, b2_l, W2_r):
  ei = edge_index.astype(jnp.int32)
  s2 = ei[0:1] * 2
  idx = jnp.concatenate([s2, s2 + 1, ei[1:2]], axis=0).reshape(
      3, NS, CPT, CHUNK)
  xr = x.reshape(2 * N_NODES, DH)

  p1, cnt = _sc_agg_counts(xr, idx)
  h = _dense(p1, cnt, x, W1_l, b1_l.reshape(1, D), W1_r)
  (p2,) = _sc_agg(h.reshape(2 * N_NODES, DH), idx)
  return _dense(p2, cnt, h, W2_l, b2_l.reshape(1, D), W2_r)
